# baseline (speedup 1.0000x reference)
import functools

import jax
import jax.numpy as jnp
from jax import lax
from jax.experimental import pallas as pl
from jax.experimental.pallas import tpu as pltpu
from jax.experimental.pallas import tpu_sc as plsc


def kernel(input, unit_mapping):
    B, C, H, W = input.shape
    info = plsc.get_sparse_core_info()
    num_cores = info.num_cores
    nw = info.num_cores * info.num_subcores
    planes = B * C
    per_w = planes // nw
    mesh = plsc.VectorSubcoreMesh(core_axis_name="c", subcore_axis_name="s")

    mask16 = jnp.broadcast_to(unit_mapping[0][:, None], (C, 16))

    @functools.partial(
        pl.kernel,
        mesh=mesh,
        out_type=jax.ShapeDtypeStruct((B, C, H, W), jnp.float32),
        scratch_types=[
            pltpu.VMEM((H, W), jnp.float32),
            pltpu.VMEM((16,), jnp.float32),
        ],
    )
    def sc_mul(x_hbm, m_hbm, o_hbm, plane_v, s_v):
        wid = lax.axis_index("s") * num_cores + lax.axis_index("c")

        def plane_body(i, carry):
            p = wid * per_w + i
            b = p // C
            c = p % C
            pltpu.sync_copy(x_hbm.at[b, c], plane_v)
            pltpu.sync_copy(m_hbm.at[c], s_v)
            s = s_v[...]

            def row_body(r, carry2):
                def col_body(j, carry3):
                    v = plane_v[r, pl.ds(j * 16, 16)]
                    plane_v[r, pl.ds(j * 16, 16)] = v * s
                    return carry3

                return lax.fori_loop(0, W // 16, col_body, carry2)

            lax.fori_loop(0, H, row_body, 0)
            pltpu.sync_copy(plane_v, o_hbm.at[b, c])
            return carry

        lax.fori_loop(0, per_w, plane_body, 0)

    return sc_mul(input, mask16)


# speedup vs baseline: 1.8160x; 1.8160x over previous
import jax
import jax.numpy as jnp
from jax.experimental import pallas as pl
from jax.experimental.pallas import tpu as pltpu


_CB = 48


def _mul_kernel(um_ref, x_ref, o_ref):
    c0 = pl.program_id(1) * _CB
    for i in range(_CB):
        s = um_ref[0, c0 + i]
        o_ref[0, i] = x_ref[0, i] * s


def kernel(input, unit_mapping):
    B, C, H, W = input.shape
    grid = (B, C // _CB)
    out = pl.pallas_call(
        _mul_kernel,
        grid=grid,
        in_specs=[
            pl.BlockSpec(memory_space=pltpu.SMEM),
            pl.BlockSpec((1, _CB, H, W), lambda b, c: (b, c, 0, 0)),
        ],
        out_specs=pl.BlockSpec((1, _CB, H, W), lambda b, c: (b, c, 0, 0)),
        out_shape=jax.ShapeDtypeStruct((B, C, H, W), input.dtype),
        compiler_params=pltpu.CompilerParams(
            dimension_semantics=("arbitrary", "arbitrary")),
    )(unit_mapping, input)
    return out
